# Initial kernel scaffold; baseline (speedup 1.0000x reference)
import jax
import jax.numpy as jnp
from jax.experimental import pallas as pl


def kernel(x0, edge_index, edge_weight, W1_rel, b1, W1_root, W2_rel, b2, W2_root, W3_rel, b3, W3_root, W_lin, b_lin):
    raise NotImplementedError("write your pallas kernel here")


# speedup vs baseline: 3.3341x; 3.3341x over previous
import functools

import jax
import jax.numpy as jnp
from jax import lax
from jax.experimental import pallas as pl
from jax.experimental.pallas import tpu as pltpu
from jax.experimental.pallas import tpu_sc as plsc

NC = 2
NS = 16
NW = NC * NS
CH = 80
LANES = 16


def _sc_scatter(y, src, dst, w):
    n, d = y.shape
    e = src.shape[0]
    epw = e // NW
    nchunk = epw // CH
    gran = 8 * NS * 4
    n_pad = ((n + gran - 1) // gran) * gran
    rpt = n_pad // NS
    zr = rpt // 4
    mesh = plsc.VectorSubcoreMesh(core_axis_name="c", subcore_axis_name="s")

    @functools.partial(
        pl.kernel,
        out_type=jax.ShapeDtypeStruct((NC, n_pad, d), jnp.float32),
        mesh=mesh,
        compiler_params=pltpu.CompilerParams(needs_layout_passes=False),
        scratch_types=[
            pltpu.VMEM_SHARED((n_pad, d), jnp.float32),
            pltpu.VMEM((CH,), jnp.int32),
            pltpu.VMEM((CH,), jnp.int32),
            pltpu.VMEM((CH,), jnp.float32),
            pltpu.VMEM((CH, d), jnp.float32),
            pltpu.VMEM((zr, d), jnp.float32),
            pltpu.SemaphoreType.DMA,
        ],
    )
    def scatter_kernel(y_hbm, src_hbm, dst_hbm, w_hbm, out_hbm,
                       acc, src_v, dst_v, w_v, rows_v, zero_v, sem):
        c = lax.axis_index("c")
        s = lax.axis_index("s")

        def zfill(i, _):
            for j in range(d // LANES):
                zero_v[i, pl.ds(j * LANES, LANES)] = jnp.zeros((LANES,), jnp.float32)
            return 0
        lax.fori_loop(0, zr, zfill, 0)

        def zcopy(i, _):
            zoff = pl.multiple_of(s * rpt + i * zr, 8)
            pltpu.sync_copy(zero_v, acc.at[pl.ds(zoff, zr)])
            return 0
        lax.fori_loop(0, rpt // zr, zcopy, 0)
        plsc.subcore_barrier()

        base = (c * NS + s) * epw

        def chunk(kk, _):
            off = pl.multiple_of(base + kk * CH, 8)
            pltpu.sync_copy(src_hbm.at[pl.ds(off, CH)], src_v)
            pltpu.sync_copy(dst_hbm.at[pl.ds(off, CH)], dst_v)
            pltpu.sync_copy(w_hbm.at[pl.ds(off, CH)], w_v)
            pltpu.async_copy(y_hbm.at[src_v], rows_v, sem).wait()

            def scale(ee, _):
                wb = plsc.load_gather(w_v, [jnp.full((LANES,), ee, jnp.int32)])
                for j in range(d // LANES):
                    sl = pl.ds(j * LANES, LANES)
                    rows_v[ee, sl] = rows_v[ee, sl] * wb
                return 0
            lax.fori_loop(0, CH, scale, 0)

            pltpu.sync_copy(rows_v, acc.at[dst_v], add=True)
            return 0
        lax.fori_loop(0, nchunk, chunk, 0)
        plsc.subcore_barrier()

        ooff = pl.multiple_of(s * rpt, 8)
        pltpu.sync_copy(acc.at[pl.ds(ooff, rpt)],
                        out_hbm.at[c, pl.ds(ooff, rpt)])

    return scatter_kernel(y, src, dst, w)


_ROWS = 1000


def _tc_pre(x, w_rel, w_root, b):
    n, d = x.shape
    h = w_rel.shape[1]

    def body(x_ref, wr_ref, wt_ref, b_ref, y_ref, z_ref):
        xb = x_ref[...]
        y_ref[...] = jnp.dot(xb, wr_ref[...], preferred_element_type=jnp.float32)
        z_ref[...] = jnp.dot(xb, wt_ref[...], preferred_element_type=jnp.float32) + b_ref[...]

    return pl.pallas_call(
        body,
        grid=(n // _ROWS,),
        in_specs=[pl.BlockSpec((_ROWS, d), lambda i: (i, 0)),
                  pl.BlockSpec((d, h), lambda i: (0, 0)),
                  pl.BlockSpec((d, h), lambda i: (0, 0)),
                  pl.BlockSpec((1, h), lambda i: (0, 0))],
        out_specs=[pl.BlockSpec((_ROWS, h), lambda i: (i, 0)),
                   pl.BlockSpec((_ROWS, h), lambda i: (i, 0))],
        out_shape=[jax.ShapeDtypeStruct((n, h), jnp.float32)] * 2,
    )(x, w_rel, w_root, b.reshape(1, -1))


def _tc_mid(part, z_prev, w_rel, w_root, b):
    n, d = z_prev.shape
    h = w_rel.shape[1]

    def body(p_ref, zp_ref, wr_ref, wt_ref, b_ref, x_ref, y_ref, z_ref):
        xb = jnp.maximum(p_ref[0] + p_ref[1] + zp_ref[...], 0.0)
        x_ref[...] = xb
        y_ref[...] = jnp.dot(xb, wr_ref[...], preferred_element_type=jnp.float32)
        z_ref[...] = jnp.dot(xb, wt_ref[...], preferred_element_type=jnp.float32) + b_ref[...]

    return pl.pallas_call(
        body,
        grid=(n // _ROWS,),
        in_specs=[pl.BlockSpec((2, _ROWS, d), lambda i: (0, i, 0)),
                  pl.BlockSpec((_ROWS, d), lambda i: (i, 0)),
                  pl.BlockSpec((d, h), lambda i: (0, 0)),
                  pl.BlockSpec((d, h), lambda i: (0, 0)),
                  pl.BlockSpec((1, h), lambda i: (0, 0))],
        out_specs=[pl.BlockSpec((_ROWS, d), lambda i: (i, 0)),
                   pl.BlockSpec((_ROWS, h), lambda i: (i, 0)),
                   pl.BlockSpec((_ROWS, h), lambda i: (i, 0))],
        out_shape=[jax.ShapeDtypeStruct((n, d), jnp.float32),
                   jax.ShapeDtypeStruct((n, h), jnp.float32),
                   jax.ShapeDtypeStruct((n, h), jnp.float32)],
    )(part, z_prev, w_rel, w_root, b.reshape(1, -1))


def _tc_head(part, z3, x1, x2, w_lin, b_lin):
    n, d = z3.shape
    o = w_lin.shape[1]

    def body(p_ref, z_ref, x1_ref, x2_ref, wl_ref, bl_ref, out_ref):
        x3 = jnp.maximum(p_ref[0] + p_ref[1] + z_ref[...], 0.0)
        wl = wl_ref[...]
        t = (jnp.dot(x1_ref[...], wl[0:d], preferred_element_type=jnp.float32)
             + jnp.dot(x2_ref[...], wl[d:2 * d], preferred_element_type=jnp.float32)
             + jnp.dot(x3, wl[2 * d:3 * d], preferred_element_type=jnp.float32)
             + bl_ref[...])
        m = jnp.max(t, axis=-1, keepdims=True)
        u = t - m
        out_ref[...] = u - jnp.log(jnp.sum(jnp.exp(u), axis=-1, keepdims=True))

    return pl.pallas_call(
        body,
        grid=(n // _ROWS,),
        in_specs=[pl.BlockSpec((2, _ROWS, d), lambda i: (0, i, 0)),
                  pl.BlockSpec((_ROWS, d), lambda i: (i, 0)),
                  pl.BlockSpec((_ROWS, d), lambda i: (i, 0)),
                  pl.BlockSpec((_ROWS, d), lambda i: (i, 0)),
                  pl.BlockSpec((3 * d, o), lambda i: (0, 0)),
                  pl.BlockSpec((1, o), lambda i: (0, 0))],
        out_specs=pl.BlockSpec((_ROWS, o), lambda i: (i, 0)),
        out_shape=jax.ShapeDtypeStruct((n, o), jnp.float32),
    )(part, z3, x1, x2, w_lin, b_lin.reshape(1, -1))


def kernel(x0, edge_index, edge_weight, W1_rel, b1, W1_root, W2_rel, b2,
           W2_root, W3_rel, b3, W3_root, W_lin, b_lin):
    src = edge_index[0].astype(jnp.int32)
    dst = edge_index[1].astype(jnp.int32)
    w = edge_weight.astype(jnp.float32)

    y1, z1 = _tc_pre(x0, W1_rel, W1_root, b1)
    p1 = _sc_scatter(y1, src, dst, w)
    x1, y2, z2 = _tc_mid(p1, z1, W2_rel, W2_root, b2)
    p2 = _sc_scatter(y2, src, dst, w)
    x2, y3, z3 = _tc_mid(p2, z2, W3_rel, W3_root, b3)
    p3 = _sc_scatter(y3, src, dst, w)
    return _tc_head(p3, z3, x1, x2, W_lin, b_lin)


# speedup vs baseline: 4.8458x; 1.4534x over previous
import functools

import jax
import jax.numpy as jnp
from jax import lax
from jax.experimental import pallas as pl
from jax.experimental.pallas import tpu as pltpu
from jax.experimental.pallas import tpu_sc as plsc

NC = 2
NS = 16
NW = NC * NS
CH = 80
LANES = 16


NB = 3


def _sc_scatter(y, src, dst, w):
    n, d = y.shape
    e = w.shape[0]
    epw = e // NW
    nchunk = epw // CH
    gran = 8 * NS * 4
    n_pad = ((n + gran - 1) // gran) * gran
    rpt = n_pad // NS
    zr = 16
    mesh = plsc.VectorSubcoreMesh(core_axis_name="c", subcore_axis_name="s")

    @functools.partial(
        pl.kernel,
        out_type=jax.ShapeDtypeStruct((NC, n_pad, d), jnp.float32),
        mesh=mesh,
        compiler_params=pltpu.CompilerParams(needs_layout_passes=False),
    scratch_types=(
        [pltpu.VMEM_SHARED((n_pad, d), jnp.float32)]
        + [pltpu.VMEM((CH,), jnp.int32)] * NB
        + [pltpu.VMEM((CH,), jnp.int32)] * NB
        + [pltpu.VMEM((CH,), jnp.float32)] * NB
        + [pltpu.VMEM((CH, d), jnp.float32)] * NB
        + [pltpu.VMEM((zr, d), jnp.float32)]
        + [pltpu.SemaphoreType.DMA((NB,)),
           pltpu.SemaphoreType.DMA((NB,)),
           pltpu.SemaphoreType.DMA((NB,))]
    ),
    )
    def scatter_kernel(y_hbm, src_hbm, dst_hbm, w_hbm, out_hbm, acc, *rest):
        src_r = rest[0:NB]
        dst_r = rest[NB:2 * NB]
        w_r = rest[2 * NB:3 * NB]
        rows_r = rest[3 * NB:4 * NB]
        zero_v = rest[4 * NB]
        isem, gsem, ssem = rest[4 * NB + 1:4 * NB + 4]
        c = lax.axis_index("c")
        s = lax.axis_index("s")
        wid = c * NS + s
        ebase = pl.multiple_of(wid * epw, 8)

        def zfill(i, _):
            for j in range(d // LANES):
                zero_v[i, pl.ds(j * LANES, LANES)] = jnp.zeros((LANES,), jnp.float32)
            return 0
        lax.fori_loop(0, zr, zfill, 0)

        def zcopy(i, _):
            zoff = pl.multiple_of(s * rpt + i * zr, 8)
            pltpu.sync_copy(zero_v, acc.at[pl.ds(zoff, zr)])
            return 0
        lax.fori_loop(0, rpt // zr, zcopy, 0)
        plsc.subcore_barrier()

        def istart(kk, b):
            eoff = pl.multiple_of(ebase + kk * CH, 8)
            pltpu.async_copy(src_hbm.at[pl.ds(eoff, CH)], src_r[b], isem.at[b])
            pltpu.async_copy(dst_hbm.at[pl.ds(eoff, CH)], dst_r[b], isem.at[b])
            pltpu.async_copy(w_hbm.at[pl.ds(eoff, CH)], w_r[b], isem.at[b])

        def iwait(kk, b):
            eoff = pl.multiple_of(ebase + kk * CH, 8)
            pltpu.make_async_copy(src_hbm.at[pl.ds(eoff, CH)], src_r[b],
                                  isem.at[b]).wait()
            pltpu.make_async_copy(dst_hbm.at[pl.ds(eoff, CH)], dst_r[b],
                                  isem.at[b]).wait()
            pltpu.make_async_copy(w_hbm.at[pl.ds(eoff, CH)], w_r[b],
                                  isem.at[b]).wait()

        def gstart(b):
            pltpu.async_copy(y_hbm.at[src_r[b]], rows_r[b], gsem.at[b])

        def gwait(b):
            pltpu.make_async_copy(y_hbm.at[src_r[b]], rows_r[b],
                                  gsem.at[b]).wait()

        def sstart(b):
            pltpu.async_copy(rows_r[b], acc.at[dst_r[b]], ssem.at[b],
                             add=True).wait()

        def swait(b):
            pass

        def scale(b):
            rows_v = rows_r[b]
            w_v = w_r[b]

            def body(ee, _):
                wb = plsc.load_gather(
                    w_v, [jnp.full((LANES,), ee, jnp.int32)])
                for j in range(d // LANES):
                    sl = pl.ds(j * LANES, LANES)
                    rows_v[ee, sl] = rows_v[ee, sl] * wb
                return 0
            lax.fori_loop(0, CH, body, 0)

        for b in range(NB):
            istart(b, b)
        iwait(0, 0)
        gstart(0)
        gwait(0)
        scale(0)
        sstart(0)
        iwait(1, 1)

        def outer(i, _):
            m0 = i * NB + 1
            for t in range(NB):
                m = m0 + t
                b_prev = t
                b_cur = (t + 1) % NB
                b_nxt = (t + 2) % NB
                swait(b_prev)
                istart(m + NB - 1, b_prev)
                gstart(b_cur)
                gwait(b_cur)
                scale(b_cur)
                sstart(b_cur)
                iwait(m + 1, b_nxt)
            return 0
        ngroups = (nchunk - NB) // NB
        lax.fori_loop(0, ngroups, outer, 0)

        for m in range(NB * ngroups + 1, nchunk):
            b_prev = (m - 1) % NB
            b_cur = m % NB
            b_nxt = (m + 1) % NB
            swait(b_prev)
            if m + NB - 1 < nchunk:
                istart(m + NB - 1, b_prev)
            gstart(b_cur)
            gwait(b_cur)
            scale(b_cur)
            sstart(b_cur)
            if m + 1 < nchunk:
                iwait(m + 1, b_nxt)
        swait((nchunk - 1) % NB)
        plsc.subcore_barrier()

        ooff = pl.multiple_of(s * rpt, 8)
        pltpu.sync_copy(acc.at[pl.ds(ooff, rpt)],
                        out_hbm.at[c, pl.ds(ooff, rpt)])

    return scatter_kernel(y, src, dst, w)


_ROWS = 1000


def _tc_pre(x, w_rel, w_root, b):
    n, d = x.shape
    h = w_rel.shape[1]

    def body(x_ref, wr_ref, wt_ref, b_ref, y_ref, z_ref):
        xb = x_ref[...]
        y_ref[...] = jnp.dot(xb, wr_ref[...], preferred_element_type=jnp.float32)
        z_ref[...] = jnp.dot(xb, wt_ref[...], preferred_element_type=jnp.float32) + b_ref[...]

    return pl.pallas_call(
        body,
        grid=(n // _ROWS,),
        in_specs=[pl.BlockSpec((_ROWS, d), lambda i: (i, 0)),
                  pl.BlockSpec((d, h), lambda i: (0, 0)),
                  pl.BlockSpec((d, h), lambda i: (0, 0)),
                  pl.BlockSpec((1, h), lambda i: (0, 0))],
        out_specs=[pl.BlockSpec((_ROWS, h), lambda i: (i, 0)),
                   pl.BlockSpec((_ROWS, h), lambda i: (i, 0))],
        out_shape=[jax.ShapeDtypeStruct((n, h), jnp.float32)] * 2,
    )(x, w_rel, w_root, b.reshape(1, -1))


def _tc_mid(part, z_prev, w_rel, w_root, b):
    n, d = z_prev.shape
    h = w_rel.shape[1]

    def body(p_ref, zp_ref, wr_ref, wt_ref, b_ref, x_ref, y_ref, z_ref):
        xb = jnp.maximum(p_ref[0] + p_ref[1] + zp_ref[...], 0.0)
        x_ref[...] = xb
        y_ref[...] = jnp.dot(xb, wr_ref[...], preferred_element_type=jnp.float32)
        z_ref[...] = jnp.dot(xb, wt_ref[...], preferred_element_type=jnp.float32) + b_ref[...]

    return pl.pallas_call(
        body,
        grid=(n // _ROWS,),
        in_specs=[pl.BlockSpec((2, _ROWS, d), lambda i: (0, i, 0)),
                  pl.BlockSpec((_ROWS, d), lambda i: (i, 0)),
                  pl.BlockSpec((d, h), lambda i: (0, 0)),
                  pl.BlockSpec((d, h), lambda i: (0, 0)),
                  pl.BlockSpec((1, h), lambda i: (0, 0))],
        out_specs=[pl.BlockSpec((_ROWS, d), lambda i: (i, 0)),
                   pl.BlockSpec((_ROWS, h), lambda i: (i, 0)),
                   pl.BlockSpec((_ROWS, h), lambda i: (i, 0))],
        out_shape=[jax.ShapeDtypeStruct((n, d), jnp.float32),
                   jax.ShapeDtypeStruct((n, h), jnp.float32),
                   jax.ShapeDtypeStruct((n, h), jnp.float32)],
    )(part, z_prev, w_rel, w_root, b.reshape(1, -1))


def _tc_head(part, z3, x1, x2, w_lin, b_lin):
    n, d = z3.shape
    o = w_lin.shape[1]

    def body(p_ref, z_ref, x1_ref, x2_ref, wl_ref, bl_ref, out_ref):
        x3 = jnp.maximum(p_ref[0] + p_ref[1] + z_ref[...], 0.0)
        wl = wl_ref[...]
        t = (jnp.dot(x1_ref[...], wl[0:d], preferred_element_type=jnp.float32)
             + jnp.dot(x2_ref[...], wl[d:2 * d], preferred_element_type=jnp.float32)
             + jnp.dot(x3, wl[2 * d:3 * d], preferred_element_type=jnp.float32)
             + bl_ref[...])
        m = jnp.max(t, axis=-1, keepdims=True)
        u = t - m
        out_ref[...] = u - jnp.log(jnp.sum(jnp.exp(u), axis=-1, keepdims=True))

    return pl.pallas_call(
        body,
        grid=(n // _ROWS,),
        in_specs=[pl.BlockSpec((2, _ROWS, d), lambda i: (0, i, 0)),
                  pl.BlockSpec((_ROWS, d), lambda i: (i, 0)),
                  pl.BlockSpec((_ROWS, d), lambda i: (i, 0)),
                  pl.BlockSpec((_ROWS, d), lambda i: (i, 0)),
                  pl.BlockSpec((3 * d, o), lambda i: (0, 0)),
                  pl.BlockSpec((1, o), lambda i: (0, 0))],
        out_specs=pl.BlockSpec((_ROWS, o), lambda i: (i, 0)),
        out_shape=jax.ShapeDtypeStruct((n, o), jnp.float32),
    )(part, z3, x1, x2, w_lin, b_lin.reshape(1, -1))


def kernel(x0, edge_index, edge_weight, W1_rel, b1, W1_root, W2_rel, b2,
           W2_root, W3_rel, b3, W3_root, W_lin, b_lin):
    src = edge_index[0].astype(jnp.int32)
    dst = edge_index[1].astype(jnp.int32)
    w = edge_weight.astype(jnp.float32)

    y1, z1 = _tc_pre(x0, W1_rel, W1_root, b1)
    p1 = _sc_scatter(y1, src, dst, w)
    x1, y2, z2 = _tc_mid(p1, z1, W2_rel, W2_root, b2)
    p2 = _sc_scatter(y2, src, dst, w)
    x2, y3, z3 = _tc_mid(p2, z2, W3_rel, W3_root, b3)
    p3 = _sc_scatter(y3, src, dst, w)
    return _tc_head(p3, z3, x1, x2, W_lin, b_lin)


# speedup vs baseline: 4.8616x; 1.0032x over previous
import functools

import jax
import jax.numpy as jnp
from jax import lax
from jax.experimental import pallas as pl
from jax.experimental.pallas import tpu as pltpu
from jax.experimental.pallas import tpu_sc as plsc

NC = 2
NS = 16
NW = NC * NS
CH = 80
LANES = 16


NB = 3


def _sc_scatter(y, src, dst, w):
    n, d = y.shape
    e = w.shape[0]
    epw = e // NW
    nchunk = epw // CH
    gran = 8 * NS * 4
    n_pad = ((n + gran - 1) // gran) * gran
    rpt = n_pad // NS
    zr = 16
    mesh = plsc.VectorSubcoreMesh(core_axis_name="c", subcore_axis_name="s")

    @functools.partial(
        pl.kernel,
        out_type=jax.ShapeDtypeStruct((NC, n_pad, d), jnp.float32),
        mesh=mesh,
        compiler_params=pltpu.CompilerParams(needs_layout_passes=False),
    scratch_types=(
        [pltpu.VMEM_SHARED((n_pad, d), jnp.float32)]
        + [pltpu.VMEM((CH,), jnp.int32)] * NB
        + [pltpu.VMEM((CH,), jnp.int32)] * NB
        + [pltpu.VMEM((CH,), jnp.float32)] * NB
        + [pltpu.VMEM((CH, d), jnp.float32)] * NB
        + [pltpu.VMEM((zr, d), jnp.float32)]
        + [pltpu.SemaphoreType.DMA((NB,)),
           pltpu.SemaphoreType.DMA((NB,)),
           pltpu.SemaphoreType.DMA((NB,))]
    ),
    )
    def scatter_kernel(y_hbm, src_hbm, dst_hbm, w_hbm, out_hbm, acc, *rest):
        src_r = rest[0:NB]
        dst_r = rest[NB:2 * NB]
        w_r = rest[2 * NB:3 * NB]
        rows_r = rest[3 * NB:4 * NB]
        zero_v = rest[4 * NB]
        isem, gsem, ssem = rest[4 * NB + 1:4 * NB + 4]
        c = lax.axis_index("c")
        s = lax.axis_index("s")
        wid = c * NS + s
        ebase = pl.multiple_of(wid * epw, 8)

        def zfill(i, _):
            for j in range(d // LANES):
                zero_v[i, pl.ds(j * LANES, LANES)] = jnp.zeros((LANES,), jnp.float32)
            return 0
        lax.fori_loop(0, zr, zfill, 0)

        def zcopy(i, _):
            zoff = pl.multiple_of(s * rpt + i * zr, 8)
            pltpu.sync_copy(zero_v, acc.at[pl.ds(zoff, zr)])
            return 0
        lax.fori_loop(0, rpt // zr, zcopy, 0)
        plsc.subcore_barrier()

        def istart(kk, b):
            eoff = pl.multiple_of(ebase + kk * CH, 8)
            pltpu.async_copy(src_hbm.at[pl.ds(eoff, CH)], src_r[b], isem.at[b])
            pltpu.async_copy(dst_hbm.at[pl.ds(eoff, CH)], dst_r[b], isem.at[b])
            pltpu.async_copy(w_hbm.at[pl.ds(eoff, CH)], w_r[b], isem.at[b])

        def iwait(kk, b):
            eoff = pl.multiple_of(ebase + kk * CH, 8)
            pltpu.make_async_copy(src_hbm.at[pl.ds(eoff, CH)], src_r[b],
                                  isem.at[b]).wait()
            pltpu.make_async_copy(dst_hbm.at[pl.ds(eoff, CH)], dst_r[b],
                                  isem.at[b]).wait()
            pltpu.make_async_copy(w_hbm.at[pl.ds(eoff, CH)], w_r[b],
                                  isem.at[b]).wait()

        def gstart(b):
            pltpu.async_copy(y_hbm.at[src_r[b]], rows_r[b], gsem.at[b])

        def gwait(b):
            pltpu.make_async_copy(y_hbm.at[src_r[b]], rows_r[b],
                                  gsem.at[b]).wait()

        def sstart(b):
            pltpu.async_copy(rows_r[b], acc.at[dst_r[b]], ssem.at[b],
                             add=True)

        def swait(b):
            pltpu.make_async_copy(rows_r[b], acc.at[dst_r[b]],
                                  ssem.at[b]).wait()

        def scale(b):
            rows_v = rows_r[b]
            w_v = w_r[b]

            def body(ee, _):
                wb = plsc.load_gather(
                    w_v, [jnp.full((LANES,), ee, jnp.int32)])
                for j in range(d // LANES):
                    sl = pl.ds(j * LANES, LANES)
                    rows_v[ee, sl] = rows_v[ee, sl] * wb
                return 0
            lax.fori_loop(0, CH, body, 0)

        for b in range(NB):
            istart(b, b)
        iwait(0, 0)
        gstart(0)
        gwait(0)
        scale(0)
        sstart(0)
        iwait(1, 1)

        def outer(i, _):
            m0 = i * NB + 1
            for t in range(NB):
                m = m0 + t
                b_prev = t
                b_cur = (t + 1) % NB
                b_nxt = (t + 2) % NB
                swait(b_prev)
                istart(m + NB - 1, b_prev)
                gstart(b_cur)
                gwait(b_cur)
                scale(b_cur)
                sstart(b_cur)
                iwait(m + 1, b_nxt)
            return 0
        ngroups = (nchunk - NB) // NB
        lax.fori_loop(0, ngroups, outer, 0)

        for m in range(NB * ngroups + 1, nchunk):
            b_prev = (m - 1) % NB
            b_cur = m % NB
            b_nxt = (m + 1) % NB
            swait(b_prev)
            if m + NB - 1 < nchunk:
                istart(m + NB - 1, b_prev)
            gstart(b_cur)
            gwait(b_cur)
            scale(b_cur)
            sstart(b_cur)
            if m + 1 < nchunk:
                iwait(m + 1, b_nxt)
        swait((nchunk - 1) % NB)
        plsc.subcore_barrier()

        ooff = pl.multiple_of(s * rpt, 8)
        pltpu.sync_copy(acc.at[pl.ds(ooff, rpt)],
                        out_hbm.at[c, pl.ds(ooff, rpt)])

    return scatter_kernel(y, src, dst, w)


_ROWS = 1000


def _tc_pre(x, w_rel, w_root, b):
    n, d = x.shape
    h = w_rel.shape[1]

    def body(x_ref, wr_ref, wt_ref, b_ref, y_ref, z_ref):
        xb = x_ref[...]
        y_ref[...] = jnp.dot(xb, wr_ref[...], preferred_element_type=jnp.float32)
        z_ref[...] = jnp.dot(xb, wt_ref[...], preferred_element_type=jnp.float32) + b_ref[...]

    return pl.pallas_call(
        body,
        grid=(n // _ROWS,),
        in_specs=[pl.BlockSpec((_ROWS, d), lambda i: (i, 0)),
                  pl.BlockSpec((d, h), lambda i: (0, 0)),
                  pl.BlockSpec((d, h), lambda i: (0, 0)),
                  pl.BlockSpec((1, h), lambda i: (0, 0))],
        out_specs=[pl.BlockSpec((_ROWS, h), lambda i: (i, 0)),
                   pl.BlockSpec((_ROWS, h), lambda i: (i, 0))],
        out_shape=[jax.ShapeDtypeStruct((n, h), jnp.float32)] * 2,
    )(x, w_rel, w_root, b.reshape(1, -1))


def _tc_mid(part, z_prev, w_rel, w_root, b):
    n, d = z_prev.shape
    h = w_rel.shape[1]

    def body(p_ref, zp_ref, wr_ref, wt_ref, b_ref, x_ref, y_ref, z_ref):
        xb = jnp.maximum(p_ref[0] + p_ref[1] + zp_ref[...], 0.0)
        x_ref[...] = xb
        y_ref[...] = jnp.dot(xb, wr_ref[...], preferred_element_type=jnp.float32)
        z_ref[...] = jnp.dot(xb, wt_ref[...], preferred_element_type=jnp.float32) + b_ref[...]

    return pl.pallas_call(
        body,
        grid=(n // _ROWS,),
        in_specs=[pl.BlockSpec((2, _ROWS, d), lambda i: (0, i, 0)),
                  pl.BlockSpec((_ROWS, d), lambda i: (i, 0)),
                  pl.BlockSpec((d, h), lambda i: (0, 0)),
                  pl.BlockSpec((d, h), lambda i: (0, 0)),
                  pl.BlockSpec((1, h), lambda i: (0, 0))],
        out_specs=[pl.BlockSpec((_ROWS, d), lambda i: (i, 0)),
                   pl.BlockSpec((_ROWS, h), lambda i: (i, 0)),
                   pl.BlockSpec((_ROWS, h), lambda i: (i, 0))],
        out_shape=[jax.ShapeDtypeStruct((n, d), jnp.float32),
                   jax.ShapeDtypeStruct((n, h), jnp.float32),
                   jax.ShapeDtypeStruct((n, h), jnp.float32)],
    )(part, z_prev, w_rel, w_root, b.reshape(1, -1))


def _tc_head(part, z3, x1, x2, w_lin, b_lin):
    n, d = z3.shape
    o = w_lin.shape[1]

    def body(p_ref, z_ref, x1_ref, x2_ref, wl_ref, bl_ref, out_ref):
        x3 = jnp.maximum(p_ref[0] + p_ref[1] + z_ref[...], 0.0)
        wl = wl_ref[...]
        t = (jnp.dot(x1_ref[...], wl[0:d], preferred_element_type=jnp.float32)
             + jnp.dot(x2_ref[...], wl[d:2 * d], preferred_element_type=jnp.float32)
             + jnp.dot(x3, wl[2 * d:3 * d], preferred_element_type=jnp.float32)
             + bl_ref[...])
        m = jnp.max(t, axis=-1, keepdims=True)
        u = t - m
        out_ref[...] = u - jnp.log(jnp.sum(jnp.exp(u), axis=-1, keepdims=True))

    return pl.pallas_call(
        body,
        grid=(n // _ROWS,),
        in_specs=[pl.BlockSpec((2, _ROWS, d), lambda i: (0, i, 0)),
                  pl.BlockSpec((_ROWS, d), lambda i: (i, 0)),
                  pl.BlockSpec((_ROWS, d), lambda i: (i, 0)),
                  pl.BlockSpec((_ROWS, d), lambda i: (i, 0)),
                  pl.BlockSpec((3 * d, o), lambda i: (0, 0)),
                  pl.BlockSpec((1, o), lambda i: (0, 0))],
        out_specs=pl.BlockSpec((_ROWS, o), lambda i: (i, 0)),
        out_shape=jax.ShapeDtypeStruct((n, o), jnp.float32),
    )(part, z3, x1, x2, w_lin, b_lin.reshape(1, -1))


def kernel(x0, edge_index, edge_weight, W1_rel, b1, W1_root, W2_rel, b2,
           W2_root, W3_rel, b3, W3_root, W_lin, b_lin):
    src = edge_index[0].astype(jnp.int32)
    dst = edge_index[1].astype(jnp.int32)
    w = edge_weight.astype(jnp.float32)

    y1, z1 = _tc_pre(x0, W1_rel, W1_root, b1)
    p1 = _sc_scatter(y1, src, dst, w)
    x1, y2, z2 = _tc_mid(p1, z1, W2_rel, W2_root, b2)
    p2 = _sc_scatter(y2, src, dst, w)
    x2, y3, z3 = _tc_mid(p2, z2, W3_rel, W3_root, b3)
    p3 = _sc_scatter(y3, src, dst, w)
    return _tc_head(p3, z3, x1, x2, W_lin, b_lin)
